# baseline (speedup 1.0000x reference)
import functools

import jax
import jax.numpy as jnp
from jax.experimental import pallas as pl
from jax.experimental.pallas import tpu as pltpu


def _roll64(v):
    return jnp.roll(v, 64, axis=-1)


def _packed_sqnorm(xb):
    sq = xb * xb
    a, bq, cq = sq[:, 0:128], sq[:, 128:256], sq[:, 256:384]
    n_even = a + _roll64(a) + bq
    n_odd = bq + _roll64(cq) + cq
    lane = jax.lax.broadcasted_iota(jnp.int32, n_even.shape, 1)
    return jnp.where(lane < 64, n_even, n_odd)


def _norm_select_kernel(x_ref, kx_ref, norms_ref, *, m, nc, k, nh):
    c = pl.program_id(1)
    xb = x_ref[0]
    packed = _packed_sqnorm(xb)
    norms_ref[pl.ds(c * m, m), :] = jax.lax.bitcast_convert_type(packed, jnp.int32)

    @pl.when(c == nc - 1)
    def _():
        un = norms_ref[...]
        ones_row = jnp.ones((1, nh), jnp.float32)

        def body(_, carry):
            lo, hi = carry
            mid = jax.lax.shift_right_logical(lo + hi, 1)
            ind = jnp.where(un <= mid, 1.0, 0.0)
            cnt = jax.lax.dot_general(
                ones_row, ind, (((1,), (0,)), ((), ())),
                preferred_element_type=jnp.float32,
            )
            cnt = cnt + _roll64(cnt)
            pred = cnt >= float(k)
            lo2 = jnp.where(pred, lo, mid + 1)
            hi2 = jnp.where(pred, mid, hi)
            return (lo2, hi2)

        lo0 = jnp.zeros((1, 128), jnp.int32)
        hi0 = jnp.full((1, 128), jnp.int32(0x7FFFFFFF))
        lo, _ = jax.lax.fori_loop(0, 31, body, (lo0, hi0))
        kx_ref[0] = jax.lax.bitcast_convert_type(lo, jnp.float32)


def _mask_kernel(x_ref, kx_ref, o_ref):
    xb = x_ref[0]
    packed = _packed_sqnorm(xb)
    pm = (jnp.sqrt(packed) > jnp.sqrt(kx_ref[0])).astype(jnp.float32)
    lane = jax.lax.broadcasted_iota(jnp.int32, pm.shape, 1)
    pm_r = _roll64(pm)
    me = jnp.where(lane < 64, pm, pm_r)
    mo = jnp.where(lane < 64, pm_r, pm)
    a, bq, cq = xb[:, 0:128], xb[:, 128:256], xb[:, 256:384]
    o_ref[0] = jnp.concatenate([a * me, bq * pm, cq * mo], axis=-1)


def kernel(x):
    b, n, c3, d = x.shape
    assert c3 == 3 and d == 64
    k = n // 10
    nh = n // 2

    m = 1024
    nc = nh // m

    xt = x.reshape(b, nh, 384)

    kx = pl.pallas_call(
        functools.partial(_norm_select_kernel, m=m, nc=nc, k=k, nh=nh),
        grid=(b, nc),
        in_specs=[pl.BlockSpec((1, m, 384), lambda bi, ci: (bi, ci, 0))],
        out_specs=pl.BlockSpec((1, 1, 128), lambda bi, ci: (bi, 0, 0)),
        out_shape=jax.ShapeDtypeStruct((b, 1, 128), jnp.float32),
        scratch_shapes=[pltpu.VMEM((nh, 128), jnp.int32)],
    )(xt)

    out = pl.pallas_call(
        _mask_kernel,
        grid=(b, nc),
        in_specs=[
            pl.BlockSpec((1, m, 384), lambda bi, ci: (bi, ci, 0)),
            pl.BlockSpec((1, 1, 128), lambda bi, ci: (bi, 0, 0)),
        ],
        out_specs=pl.BlockSpec((1, m, 384), lambda bi, ci: (bi, ci, 0)),
        out_shape=jax.ShapeDtypeStruct((b, nh, 384), jnp.float32),
    )(xt, kx)

    return out.reshape(b, n, c3, d)


# speedup vs baseline: 1.0035x; 1.0035x over previous
import functools

import jax
import jax.numpy as jnp
from jax.experimental import pallas as pl
from jax.experimental.pallas import tpu as pltpu


def _roll64(v):
    return jnp.roll(v, 64, axis=-1)


def _packed_sqnorm(xb):
    sq = xb * xb
    a, bq, cq = sq[:, 0:128], sq[:, 128:256], sq[:, 256:384]
    n_even = a + _roll64(a) + bq
    n_odd = bq + _roll64(cq) + cq
    lane = jax.lax.broadcasted_iota(jnp.int32, n_even.shape, 1)
    return jnp.where(lane < 64, n_even, n_odd)


def _norm_select_kernel(x_ref, kx_ref, norms_ref, *, m, nc, k, nh):
    c = pl.program_id(1)
    xb = x_ref[0]
    packed = _packed_sqnorm(xb)
    norms_ref[pl.ds(c * m, m), :] = jax.lax.bitcast_convert_type(packed, jnp.int32)

    @pl.when(c == nc - 1)
    def _():
        un = norms_ref[...]
        ones_row = jnp.ones((1, nh), jnp.float32)

        def body(_, carry):
            lo, hi = carry
            mid = jax.lax.shift_right_logical(lo + hi, 1)
            ind = jnp.where(un <= mid, 1.0, 0.0)
            cnt = jax.lax.dot_general(
                ones_row, ind, (((1,), (0,)), ((), ())),
                preferred_element_type=jnp.float32,
            )
            cnt = cnt + _roll64(cnt)
            pred = cnt >= float(k)
            lo2 = jnp.where(pred, lo, mid + 1)
            hi2 = jnp.where(pred, mid, hi)
            return (lo2, hi2)

        lo0 = jnp.zeros((1, 128), jnp.int32)
        hi0 = jnp.full((1, 128), jnp.int32(0x7FFFFFFF))
        lo, _ = jax.lax.fori_loop(0, 31, body, (lo0, hi0))
        kx_ref[0] = jax.lax.bitcast_convert_type(lo, jnp.float32)


def _mask_kernel(x_ref, kx_ref, o_ref):
    xb = x_ref[0]
    packed = _packed_sqnorm(xb)
    pm = (packed > kx_ref[0]).astype(jnp.float32)
    lane = jax.lax.broadcasted_iota(jnp.int32, pm.shape, 1)
    pm_r = _roll64(pm)
    me = jnp.where(lane < 64, pm, pm_r)
    mo = jnp.where(lane < 64, pm_r, pm)
    a, bq, cq = xb[:, 0:128], xb[:, 128:256], xb[:, 256:384]
    o_ref[0] = jnp.concatenate([a * me, bq * pm, cq * mo], axis=-1)


def kernel(x):
    b, n, c3, d = x.shape
    assert c3 == 3 and d == 64
    k = n // 10
    nh = n // 2

    m = 1024
    nc = nh // m

    xt = x.reshape(b, nh, 384)

    kx = pl.pallas_call(
        functools.partial(_norm_select_kernel, m=m, nc=nc, k=k, nh=nh),
        grid=(b, nc),
        in_specs=[pl.BlockSpec((1, m, 384), lambda bi, ci: (bi, ci, 0))],
        out_specs=pl.BlockSpec((1, 1, 128), lambda bi, ci: (bi, 0, 0)),
        out_shape=jax.ShapeDtypeStruct((b, 1, 128), jnp.float32),
        scratch_shapes=[pltpu.VMEM((nh, 128), jnp.int32)],
    )(xt)

    out = pl.pallas_call(
        _mask_kernel,
        grid=(b, nc),
        in_specs=[
            pl.BlockSpec((1, m, 384), lambda bi, ci: (bi, ci, 0)),
            pl.BlockSpec((1, 1, 128), lambda bi, ci: (bi, 0, 0)),
        ],
        out_specs=pl.BlockSpec((1, m, 384), lambda bi, ci: (bi, ci, 0)),
        out_shape=jax.ShapeDtypeStruct((b, nh, 384), jnp.float32),
    )(xt, kx)

    return out.reshape(b, n, c3, d)


# speedup vs baseline: 2.9749x; 2.9645x over previous
import functools

import jax
import jax.numpy as jnp
from jax.experimental import pallas as pl
from jax.experimental.pallas import tpu as pltpu


def _sqnorm(xb):
    p0, p1, p2 = xb[0], xb[1], xb[2]
    return p0 * p0 + p1 * p1 + p2 * p2


def _norm_select_kernel(x_ref, kx_ref, norms_ref, *, nbn, ncn, k, n, d):
    c = pl.program_id(1)
    n2 = _sqnorm(x_ref[0])
    norms_ref[:, pl.ds(c * nbn, nbn)] = jax.lax.bitcast_convert_type(n2, jnp.int32)

    @pl.when(c == ncn - 1)
    def _():
        un = norms_ref[...]
        ones_col = jnp.ones((n, 1), jnp.float32)

        def body(_, carry):
            lo, hi = carry
            mid = jax.lax.shift_right_logical(lo + hi, 1)
            ind = jnp.where(un <= mid, 1.0, 0.0)
            cnt = jax.lax.dot_general(
                ind, ones_col, (((1,), (0,)), ((), ())),
                preferred_element_type=jnp.float32,
            )
            pred = cnt >= float(k)
            lo2 = jnp.where(pred, lo, mid + 1)
            hi2 = jnp.where(pred, mid, hi)
            return (lo2, hi2)

        lo0 = jnp.zeros((d, 1), jnp.int32)
        hi0 = jnp.full((d, 1), jnp.int32(0x7FFFFFFF))
        lo, _ = jax.lax.fori_loop(0, 31, body, (lo0, hi0))
        kxf = jax.lax.bitcast_convert_type(lo, jnp.float32)
        kx_ref[0] = jnp.broadcast_to(kxf, (d, 128))


def _mask_kernel(x_ref, kx_ref, o_ref):
    xb = x_ref[0]
    n2 = _sqnorm(xb)
    m = (n2 > kx_ref[0][:, 0:1]).astype(jnp.float32)
    o_ref[0, 0] = xb[0] * m
    o_ref[0, 1] = xb[1] * m
    o_ref[0, 2] = xb[2] * m


def kernel(x):
    b, n, c3, d = x.shape
    assert c3 == 3
    k = n // 10

    nbn = 2048
    ncn = n // nbn

    xp = jnp.transpose(x, (0, 2, 3, 1))

    kx = pl.pallas_call(
        functools.partial(_norm_select_kernel, nbn=nbn, ncn=ncn, k=k, n=n, d=d),
        grid=(b, ncn),
        in_specs=[pl.BlockSpec((1, c3, d, nbn), lambda bi, ci: (bi, 0, 0, ci))],
        out_specs=pl.BlockSpec((1, d, 128), lambda bi, ci: (bi, 0, 0)),
        out_shape=jax.ShapeDtypeStruct((b, d, 128), jnp.float32),
        scratch_shapes=[pltpu.VMEM((d, n), jnp.int32)],
    )(xp)

    outp = pl.pallas_call(
        _mask_kernel,
        grid=(b, ncn),
        in_specs=[
            pl.BlockSpec((1, c3, d, nbn), lambda bi, ci: (bi, 0, 0, ci)),
            pl.BlockSpec((1, d, 128), lambda bi, ci: (bi, 0, 0)),
        ],
        out_specs=pl.BlockSpec((1, c3, d, nbn), lambda bi, ci: (bi, 0, 0, ci)),
        out_shape=jax.ShapeDtypeStruct((b, c3, d, n), jnp.float32),
    )(xp, kx)

    return jnp.transpose(outp, (0, 3, 1, 2))


# speedup vs baseline: 3.1727x; 1.0665x over previous
import functools

import jax
import jax.numpy as jnp
from jax.experimental import pallas as pl
from jax.experimental.pallas import tpu as pltpu


def _sqnorm(xb):
    p0, p1, p2 = xb[0], xb[1], xb[2]
    return p0 * p0 + p1 * p1 + p2 * p2


def _fused_kernel(xn_ref, xm_ref, o_ref, norms_ref, kxs_ref, *, nbn, ncn, k, n, d, nb):
    bi = pl.program_id(0)
    ci = pl.program_id(1)

    @pl.when(bi < nb)
    def _():
        n2 = _sqnorm(xn_ref[0])
        norms_ref[:, pl.ds(ci * nbn, nbn)] = jax.lax.bitcast_convert_type(
            n2, jnp.int32
        )

    @pl.when(bi > 0)
    def _():
        xb = xm_ref[0]
        n2 = _sqnorm(xb)
        m = (n2 > kxs_ref[:, 0:1]).astype(jnp.float32)
        o_ref[0, 0] = xb[0] * m
        o_ref[0, 1] = xb[1] * m
        o_ref[0, 2] = xb[2] * m

    @pl.when((bi < nb) & (ci == ncn - 1))
    def _():
        un = norms_ref[...]
        ones_col = jnp.ones((n, 1), jnp.float32)

        def body(_, carry):
            lo, hi = carry
            mid = jax.lax.shift_right_logical(lo + hi, 1)
            ind = jnp.where(un <= mid, 1.0, 0.0)
            cnt = jax.lax.dot_general(
                ind, ones_col, (((1,), (0,)), ((), ())),
                preferred_element_type=jnp.float32,
            )
            pred = cnt >= float(k)
            lo2 = jnp.where(pred, lo, mid + 1)
            hi2 = jnp.where(pred, mid, hi)
            return (lo2, hi2)

        lo0 = jnp.zeros((d, 1), jnp.int32)
        hi0 = jnp.full((d, 1), jnp.int32(0x7FFFFFFF))
        lo, _ = jax.lax.fori_loop(0, 31, body, (lo0, hi0))
        kxf = jax.lax.bitcast_convert_type(lo, jnp.float32)
        kxs_ref[...] = jnp.broadcast_to(kxf, (d, 128))


def kernel(x):
    b, n, c3, d = x.shape
    assert c3 == 3
    k = n // 10

    nbn = 2048
    ncn = n // nbn

    xp = jnp.transpose(x, (0, 2, 3, 1))

    outp = pl.pallas_call(
        functools.partial(
            _fused_kernel, nbn=nbn, ncn=ncn, k=k, n=n, d=d, nb=b
        ),
        grid=(b + 1, ncn),
        in_specs=[
            pl.BlockSpec(
                (1, c3, d, nbn),
                lambda bi, ci, _b=b: (jnp.minimum(bi, _b - 1), 0, 0, ci),
            ),
            pl.BlockSpec(
                (1, c3, d, nbn),
                lambda bi, ci: (jnp.maximum(bi - 1, 0), 0, 0, ci),
            ),
        ],
        out_specs=pl.BlockSpec(
            (1, c3, d, nbn),
            lambda bi, ci: (jnp.maximum(bi - 1, 0), 0, 0, ci),
        ),
        out_shape=jax.ShapeDtypeStruct((b, c3, d, n), jnp.float32),
        scratch_shapes=[
            pltpu.VMEM((d, n), jnp.int32),
            pltpu.VMEM((d, 128), jnp.float32),
        ],
    )(xp, xp)

    return jnp.transpose(outp, (0, 3, 1, 2))


# speedup vs baseline: 3.3144x; 1.0447x over previous
import functools

import jax
import jax.numpy as jnp
from jax.experimental import pallas as pl
from jax.experimental.pallas import tpu as pltpu


def _sqnorm(xb):
    p0, p1, p2 = xb[0], xb[1], xb[2]
    return p0 * p0 + p1 * p1 + p2 * p2


def _fused_kernel(xn_ref, xm_ref, o_ref, norms_ref, kxs_ref, *, nbn, ncn, k, n, d, nb):
    bi = pl.program_id(0)
    ci = pl.program_id(1)

    @pl.when(bi < nb)
    def _():
        n2 = _sqnorm(xn_ref[0])
        norms_ref[:, pl.ds(ci * nbn, nbn)] = jax.lax.bitcast_convert_type(
            n2, jnp.int32
        )

    @pl.when(bi > 0)
    def _():
        xb = xm_ref[0]
        n2 = _sqnorm(xb)
        m = (n2 > kxs_ref[:, 0:1]).astype(jnp.float32)
        o_ref[0, 0] = xb[0] * m
        o_ref[0, 1] = xb[1] * m
        o_ref[0, 2] = xb[2] * m

    @pl.when((bi < nb) & (ci == ncn - 1))
    def _():
        un = norms_ref[...]
        ones_col = jnp.ones((n, 1), jnp.float32)

        def body(_, carry):
            lo, hi = carry
            mid = jax.lax.shift_right_logical(lo + hi, 1)
            ind = jnp.where(un <= mid, 1.0, 0.0)
            cnt = jax.lax.dot_general(
                ind, ones_col, (((1,), (0,)), ((), ())),
                preferred_element_type=jnp.float32,
            )
            pred = cnt >= float(k)
            lo2 = jnp.where(pred, lo, mid + 1)
            hi2 = jnp.where(pred, mid, hi)
            return (lo2, hi2)

        lo0 = jnp.zeros((d, 1), jnp.int32)
        hi0 = jnp.full((d, 1), jnp.int32(0x7FFFFFFF))
        lo, _ = jax.lax.fori_loop(0, 31, body, (lo0, hi0))
        kxf = jax.lax.bitcast_convert_type(lo, jnp.float32)
        kxs_ref[...] = jnp.broadcast_to(kxf, (d, 128))


def kernel(x):
    b, n, c3, d = x.shape
    assert c3 == 3
    k = n // 10

    nbn = 4096
    ncn = n // nbn

    xp = jnp.transpose(x, (0, 2, 3, 1))

    outp = pl.pallas_call(
        functools.partial(
            _fused_kernel, nbn=nbn, ncn=ncn, k=k, n=n, d=d, nb=b
        ),
        grid=(b + 1, ncn),
        in_specs=[
            pl.BlockSpec(
                (1, c3, d, nbn),
                lambda bi, ci, _b=b: (jnp.minimum(bi, _b - 1), 0, 0, ci),
            ),
            pl.BlockSpec(
                (1, c3, d, nbn),
                lambda bi, ci: (jnp.maximum(bi - 1, 0), 0, 0, ci),
            ),
        ],
        out_specs=pl.BlockSpec(
            (1, c3, d, nbn),
            lambda bi, ci: (jnp.maximum(bi - 1, 0), 0, 0, ci),
        ),
        out_shape=jax.ShapeDtypeStruct((b, c3, d, n), jnp.float32),
        scratch_shapes=[
            pltpu.VMEM((d, n), jnp.int32),
            pltpu.VMEM((d, 128), jnp.float32),
        ],
    )(xp, xp)

    return jnp.transpose(outp, (0, 3, 1, 2))


# speedup vs baseline: 3.3940x; 1.0240x over previous
import functools

import jax
import jax.numpy as jnp
from jax.experimental import pallas as pl
from jax.experimental.pallas import tpu as pltpu


def _sqnorm(xb):
    p0, p1, p2 = xb[0], xb[1], xb[2]
    return p0 * p0 + p1 * p1 + p2 * p2


def _fused_kernel(xn_ref, xm_ref, o_ref, norms_ref, kxs_ref, *, nbn, ncn, k, n, d, nb):
    bi = pl.program_id(0)
    ci = pl.program_id(1)

    @pl.when(bi < nb)
    def _():
        n2 = _sqnorm(xn_ref[0])
        norms_ref[:, pl.ds(ci * nbn, nbn)] = jax.lax.bitcast_convert_type(
            n2, jnp.int32
        )

    @pl.when(bi > 0)
    def _():
        xb = xm_ref[0]
        n2 = _sqnorm(xb)
        m = (n2 > kxs_ref[:, 0:1]).astype(jnp.float32)
        o_ref[0, 0] = xb[0] * m
        o_ref[0, 1] = xb[1] * m
        o_ref[0, 2] = xb[2] * m

    @pl.when((bi < nb) & (ci == ncn - 1))
    def _():
        un = norms_ref[...]
        ones_col = jnp.ones((n, 1), jnp.float32)

        def body(_, carry):
            lo, hi = carry
            mid = jax.lax.shift_right_logical(lo + hi, 1)
            ind = jnp.where(un <= mid, 1.0, 0.0)
            cnt = jax.lax.dot_general(
                ind, ones_col, (((1,), (0,)), ((), ())),
                preferred_element_type=jnp.float32,
            )
            pred = cnt >= float(k)
            lo2 = jnp.where(pred, lo, mid + 1)
            hi2 = jnp.where(pred, mid, hi)
            return (lo2, hi2)

        lo0 = jnp.zeros((d, 1), jnp.int32)
        hi0 = jnp.full((d, 1), jnp.int32(0x7FFFFFFF))
        lo, _ = jax.lax.fori_loop(0, 31, body, (lo0, hi0))
        kxf = jax.lax.bitcast_convert_type(lo, jnp.float32)
        kxs_ref[...] = jnp.broadcast_to(kxf, (d, 128))


def kernel(x):
    b, n, c3, d = x.shape
    assert c3 == 3
    k = n // 10

    nbn = 8192
    ncn = n // nbn

    xp = jnp.transpose(x, (0, 2, 3, 1))

    outp = pl.pallas_call(
        functools.partial(
            _fused_kernel, nbn=nbn, ncn=ncn, k=k, n=n, d=d, nb=b
        ),
        grid=(b + 1, ncn),
        in_specs=[
            pl.BlockSpec(
                (1, c3, d, nbn),
                lambda bi, ci, _b=b: (jnp.minimum(bi, _b - 1), 0, 0, ci),
            ),
            pl.BlockSpec(
                (1, c3, d, nbn),
                lambda bi, ci: (jnp.maximum(bi - 1, 0), 0, 0, ci),
            ),
        ],
        out_specs=pl.BlockSpec(
            (1, c3, d, nbn),
            lambda bi, ci: (jnp.maximum(bi - 1, 0), 0, 0, ci),
        ),
        out_shape=jax.ShapeDtypeStruct((b, c3, d, n), jnp.float32),
        scratch_shapes=[
            pltpu.VMEM((d, n), jnp.int32),
            pltpu.VMEM((d, 128), jnp.float32),
        ],
    )(xp, xp)

    return jnp.transpose(outp, (0, 3, 1, 2))
